# baseline (speedup 1.0000x reference)
import functools

import jax
import jax.numpy as jnp
from jax import lax
from jax.experimental import pallas as pl
from jax.experimental.pallas import tpu as pltpu
from jax.experimental.pallas import tpu_sc as plsc

P_LEN = 96
C_DIM = 128
ROWS = 3312
BLK = 207
HEAD_BLOCKS = 3
TAIL_BLOCKS = ROWS // BLK - HEAD_BLOCKS

_NW = 12
_ROWS_PER_W = P_LEN // _NW


@functools.lru_cache(maxsize=1)
def _make_sc_gather():
    mesh = plsc.VectorSubcoreMesh(core_axis_name="c", subcore_axis_name="s")

    @functools.partial(
        pl.kernel,
        mesh=mesh,
        out_type=jax.ShapeDtypeStruct((P_LEN, C_DIM), jnp.float32),
        scratch_types=[
            pltpu.VMEM((_ROWS_PER_W,), jnp.int32),
            pltpu.VMEM((_ROWS_PER_W, C_DIM), jnp.float32),
            pltpu.SemaphoreType.DMA,
        ],
    )
    def gather_kernel(idx_hbm, table_hbm, out_hbm, idx_v, rows_v, sem):
        wid = lax.axis_index("s") * 2 + lax.axis_index("c")

        @pl.when(wid < _NW)
        def _():
            base = wid * _ROWS_PER_W
            pltpu.sync_copy(idx_hbm.at[pl.ds(base, _ROWS_PER_W)], idx_v)
            pltpu.async_copy(table_hbm.at[idx_v], rows_v, sem).wait()
            pltpu.sync_copy(rows_v, out_hbm.at[pl.ds(base, _ROWS_PER_W)])

    return gather_kernel


def _tc1_body(idx_smem, x_ref, table_ref, o_ref, pe_scr):
    @pl.when(pl.program_id(0) == 0)
    def _():
        def gather_row(i, carry):
            r = idx_smem[i]
            pe_scr[pl.ds(i, 1), :] = table_ref[pl.ds(r, 1), :]
            return carry

        lax.fori_loop(0, P_LEN, gather_row, 0)

    o_ref[...] = x_ref[...] + pe_scr[...]


def _tc2_body(y_hbm, x_ref, pe_ref, o_ref):
    del y_hbm
    o_ref[...] = x_ref[...] + pe_ref[...]


def kernel(input_data, index, position_embedding):
    B, N, P, C = input_data.shape
    x = input_data.reshape(B * N, P, C)
    idx32 = index.astype(jnp.int32)

    pe_rows = _make_sc_gather()(idx32, position_embedding)

    y1 = pl.pallas_call(
        _tc1_body,
        grid=(HEAD_BLOCKS,),
        in_specs=[
            pl.BlockSpec(memory_space=pltpu.SMEM),
            pl.BlockSpec((BLK, P, C), lambda i: (i, 0, 0)),
            pl.BlockSpec((position_embedding.shape[0], C), lambda i: (0, 0)),
        ],
        out_specs=pl.BlockSpec((BLK, P, C), lambda i: (i, 0, 0)),
        out_shape=jax.ShapeDtypeStruct((B * N, P, C), jnp.float32),
        scratch_shapes=[pltpu.VMEM((P, C), jnp.float32)],
        compiler_params=pltpu.CompilerParams(
            dimension_semantics=("arbitrary",),
        ),
    )(idx32, x, position_embedding)

    out = pl.pallas_call(
        _tc2_body,
        grid=(TAIL_BLOCKS,),
        in_specs=[
            pl.BlockSpec(memory_space=pl.ANY),
            pl.BlockSpec((BLK, P, C), lambda j: (j + HEAD_BLOCKS, 0, 0)),
            pl.BlockSpec((P, C), lambda j: (0, 0)),
        ],
        out_specs=pl.BlockSpec((BLK, P, C), lambda j: (j + HEAD_BLOCKS, 0, 0)),
        out_shape=jax.ShapeDtypeStruct((B * N, P, C), jnp.float32),
        input_output_aliases={0: 0},
        compiler_params=pltpu.CompilerParams(
            dimension_semantics=("arbitrary",),
        ),
    )(y1, x, pe_rows)

    return out.reshape(B, N, P, C)


# speedup vs baseline: 1.0147x; 1.0147x over previous
import functools

import jax
import jax.numpy as jnp
from jax import lax
from jax.experimental import pallas as pl
from jax.experimental.pallas import tpu as pltpu
from jax.experimental.pallas import tpu_sc as plsc

P_LEN = 96
C_DIM = 128
ROWS = 3312
BLK = 207
HEAD_BLOCKS = 3
TAIL_BLOCKS = ROWS // BLK - HEAD_BLOCKS

_NW = 12
_ROWS_PER_W = P_LEN // _NW


@functools.lru_cache(maxsize=1)
def _make_sc_gather():
    mesh = plsc.VectorSubcoreMesh(
        core_axis_name="c", subcore_axis_name="s", num_cores=1
    )

    @functools.partial(
        pl.kernel,
        mesh=mesh,
        out_type=jax.ShapeDtypeStruct((P_LEN, C_DIM), jnp.float32),
        scratch_types=[
            pltpu.VMEM((_ROWS_PER_W,), jnp.int32),
            pltpu.VMEM((_ROWS_PER_W, C_DIM), jnp.float32),
            pltpu.SemaphoreType.DMA,
        ],
    )
    def gather_kernel(idx_hbm, table_hbm, out_hbm, idx_v, rows_v, sem):
        wid = lax.axis_index("s")

        @pl.when(wid < _NW)
        def _():
            base = wid * _ROWS_PER_W
            pltpu.sync_copy(idx_hbm.at[pl.ds(base, _ROWS_PER_W)], idx_v)
            pltpu.async_copy(table_hbm.at[idx_v], rows_v, sem).wait()
            pltpu.sync_copy(rows_v, out_hbm.at[pl.ds(base, _ROWS_PER_W)])

    return gather_kernel


def _tc1_body(idx_smem, x_ref, table_ref, o_ref, pe_scr):
    @pl.when(pl.program_id(0) == 0)
    def _():
        def gather_row(i, carry):
            r = idx_smem[i]
            pe_scr[pl.ds(i, 1), :] = table_ref[pl.ds(r, 1), :]
            return carry

        lax.fori_loop(0, P_LEN, gather_row, 0)

    o_ref[...] = x_ref[...] + pe_scr[...]


def _tc2_body(y_hbm, x_ref, pe_ref, o_ref):
    del y_hbm
    o_ref[...] = x_ref[...] + pe_ref[...]


def kernel(input_data, index, position_embedding):
    B, N, P, C = input_data.shape
    x = input_data.reshape(B * N, P, C)
    idx32 = index.astype(jnp.int32)

    pe_rows = _make_sc_gather()(idx32, position_embedding)

    y1 = pl.pallas_call(
        _tc1_body,
        grid=(HEAD_BLOCKS,),
        in_specs=[
            pl.BlockSpec(memory_space=pltpu.SMEM),
            pl.BlockSpec((BLK, P, C), lambda i: (i, 0, 0)),
            pl.BlockSpec((position_embedding.shape[0], C), lambda i: (0, 0)),
        ],
        out_specs=pl.BlockSpec((BLK, P, C), lambda i: (i, 0, 0)),
        out_shape=jax.ShapeDtypeStruct((B * N, P, C), jnp.float32),
        scratch_shapes=[pltpu.VMEM((P, C), jnp.float32)],
        compiler_params=pltpu.CompilerParams(
            dimension_semantics=("arbitrary",),
        ),
    )(idx32, x, position_embedding)

    out = pl.pallas_call(
        _tc2_body,
        grid=(TAIL_BLOCKS,),
        in_specs=[
            pl.BlockSpec(memory_space=pl.ANY),
            pl.BlockSpec((BLK, P, C), lambda j: (j + HEAD_BLOCKS, 0, 0)),
            pl.BlockSpec((P, C), lambda j: (0, 0)),
        ],
        out_specs=pl.BlockSpec((BLK, P, C), lambda j: (j + HEAD_BLOCKS, 0, 0)),
        out_shape=jax.ShapeDtypeStruct((B * N, P, C), jnp.float32),
        input_output_aliases={0: 0},
        compiler_params=pltpu.CompilerParams(
            dimension_semantics=("arbitrary",),
        ),
    )(y1, x, pe_rows)

    return out.reshape(B, N, P, C)


# speedup vs baseline: 1.0152x; 1.0004x over previous
import functools

import jax
import jax.numpy as jnp
from jax import lax
from jax.experimental import pallas as pl
from jax.experimental.pallas import tpu as pltpu
from jax.experimental.pallas import tpu_sc as plsc

P_LEN = 96
C_DIM = 128
ROWS = 3312
BLK = 207
HEAD_BLOCKS = 3
TAIL_BLOCKS = ROWS // BLK - HEAD_BLOCKS

_NW = 12
_ROWS_PER_W = P_LEN // _NW


@functools.lru_cache(maxsize=1)
def _make_sc_gather():
    mesh = plsc.VectorSubcoreMesh(
        core_axis_name="c", subcore_axis_name="s", num_cores=1, num_subcores=_NW
    )

    @functools.partial(
        pl.kernel,
        mesh=mesh,
        out_type=jax.ShapeDtypeStruct((P_LEN, C_DIM), jnp.float32),
        scratch_types=[
            pltpu.VMEM((_ROWS_PER_W,), jnp.int32),
            pltpu.VMEM((_ROWS_PER_W, C_DIM), jnp.float32),
            pltpu.SemaphoreType.DMA,
        ],
    )
    def gather_kernel(idx_hbm, table_hbm, out_hbm, idx_v, rows_v, sem):
        wid = lax.axis_index("s")

        @pl.when(wid < _NW)
        def _():
            base = wid * _ROWS_PER_W
            pltpu.sync_copy(idx_hbm.at[pl.ds(base, _ROWS_PER_W)], idx_v)
            pltpu.async_copy(table_hbm.at[idx_v], rows_v, sem).wait()
            pltpu.sync_copy(rows_v, out_hbm.at[pl.ds(base, _ROWS_PER_W)])

    return gather_kernel


def _tc1_body(idx_smem, x_ref, table_ref, o_ref, pe_scr):
    @pl.when(pl.program_id(0) == 0)
    def _():
        def gather_row(i, carry):
            r = idx_smem[i]
            pe_scr[pl.ds(i, 1), :] = table_ref[pl.ds(r, 1), :]
            return carry

        lax.fori_loop(0, P_LEN, gather_row, 0)

    o_ref[...] = x_ref[...] + pe_scr[...]


def _tc2_body(y_hbm, x_ref, pe_ref, o_ref):
    del y_hbm
    o_ref[...] = x_ref[...] + pe_ref[...]


def kernel(input_data, index, position_embedding):
    B, N, P, C = input_data.shape
    x = input_data.reshape(B * N, P, C)
    idx32 = index.astype(jnp.int32)

    pe_rows = _make_sc_gather()(idx32, position_embedding)

    y1 = pl.pallas_call(
        _tc1_body,
        grid=(HEAD_BLOCKS,),
        in_specs=[
            pl.BlockSpec(memory_space=pltpu.SMEM),
            pl.BlockSpec((BLK, P, C), lambda i: (i, 0, 0)),
            pl.BlockSpec((position_embedding.shape[0], C), lambda i: (0, 0)),
        ],
        out_specs=pl.BlockSpec((BLK, P, C), lambda i: (i, 0, 0)),
        out_shape=jax.ShapeDtypeStruct((B * N, P, C), jnp.float32),
        scratch_shapes=[pltpu.VMEM((P, C), jnp.float32)],
        compiler_params=pltpu.CompilerParams(
            dimension_semantics=("arbitrary",),
        ),
    )(idx32, x, position_embedding)

    out = pl.pallas_call(
        _tc2_body,
        grid=(TAIL_BLOCKS,),
        in_specs=[
            pl.BlockSpec(memory_space=pl.ANY),
            pl.BlockSpec((BLK, P, C), lambda j: (j + HEAD_BLOCKS, 0, 0)),
            pl.BlockSpec((P, C), lambda j: (0, 0)),
        ],
        out_specs=pl.BlockSpec((BLK, P, C), lambda j: (j + HEAD_BLOCKS, 0, 0)),
        out_shape=jax.ShapeDtypeStruct((B * N, P, C), jnp.float32),
        input_output_aliases={0: 0},
        compiler_params=pltpu.CompilerParams(
            dimension_semantics=("arbitrary",),
        ),
    )(y1, x, pe_rows)

    return out.reshape(B, N, P, C)
